# Initial kernel scaffold; baseline (speedup 1.0000x reference)
import jax
import jax.numpy as jnp
from jax.experimental import pallas as pl


def kernel(x, incident_matrix, ln_pre_g, ln_pre_b, lin1_W, lin1_b, ln1_g, ln1_b, conv1_W, conv1_b, conv2_W, conv2_b, ln2_g, ln2_b, lin2_W, lin2_b):
    raise NotImplementedError("write your pallas kernel here")


# speedup vs baseline: 68.5883x; 68.5883x over previous
import functools

import jax
import jax.numpy as jnp
from jax import lax
from jax.experimental import pallas as pl
from jax.experimental.pallas import tpu as pltpu
from jax.experimental.pallas import tpu_sc as plsc


def _tc1_body(x_ref, lpg_ref, lpb_ref, w1t_ref, b1_ref, l1g_ref, l1b_ref,
              c1wt_ref, o_ref):
    xb = x_ref[0]
    mu = jnp.mean(xb, axis=-1, keepdims=True)
    var = jnp.mean((xb - mu) ** 2, axis=-1, keepdims=True)
    y = (xb - mu) / jnp.sqrt(var + 1e-5) * lpg_ref[0] + lpb_ref[0]
    y = jnp.maximum(y, 0.0)
    z = jnp.dot(y, w1t_ref[...], preferred_element_type=jnp.float32) + b1_ref[0]
    mu2 = jnp.mean(z, axis=-1, keepdims=True)
    var2 = jnp.mean((z - mu2) ** 2, axis=-1, keepdims=True)
    z = (z - mu2) / jnp.sqrt(var2 + 1e-5) * l1g_ref[0] + l1b_ref[0]
    z = jnp.maximum(z, 0.0)
    o_ref[0] = jnp.dot(z, c1wt_ref[...], preferred_element_type=jnp.float32)


def _tc2_body(w_ref, x_ref, c2wt_ref, b2_ref, l2g_ref, l2b_ref, w2t_ref,
              b2l_ref, o_ref):
    wb = w_ref[0]
    o2 = jnp.dot(wb, c2wt_ref[...], preferred_element_type=jnp.float32) + b2_ref[0]
    mu = jnp.mean(o2, axis=-1, keepdims=True)
    var = jnp.mean((o2 - mu) ** 2, axis=-1, keepdims=True)
    t = (o2 - mu) / jnp.sqrt(var + 1e-5) * l2g_ref[0] + l2b_ref[0]
    t = jnp.maximum(t, 0.0)
    y = jnp.dot(t, w2t_ref[...], preferred_element_type=jnp.float32) + b2l_ref[0]
    o_ref[0] = x_ref[0] + y


_NS = 16
_NC = 2
_CHUNK = 128
_RB = 64


def _make_sc_kernel(N, F, nch, rows_pad):
    chunks_per_tile = rows_pad // (_NS * _RB)
    mesh = plsc.VectorSubcoreMesh(core_axis_name="c", subcore_axis_name="s")

    @functools.partial(
        pl.kernel,
        out_type=(jax.ShapeDtypeStruct((_NC, rows_pad, F), jnp.float32),
                  jax.ShapeDtypeStruct((_NC, rows_pad, F), jnp.float32)),
        mesh=mesh,
        compiler_params=pltpu.CompilerParams(
            use_tc_tiling_on_sc=False, needs_layout_passes=False),
        scratch_types=[
            pltpu.VMEM((nch, _CHUNK), jnp.int32),
            pltpu.VMEM((nch, _CHUNK), jnp.int32),
            pltpu.VMEM((nch, _CHUNK), jnp.int32),
            pltpu.VMEM((nch, _CHUNK), jnp.int32),
            pltpu.VMEM((_CHUNK, F), jnp.float32),
            pltpu.VMEM((_RB, F), jnp.float32),
            pltpu.VMEM((rows_pad // _NS,), jnp.float32),
            pltpu.VMEM((rows_pad // _NS,), jnp.float32),
            pltpu.VMEM((_RB, F), jnp.float32),
            pltpu.VMEM((_CHUNK, F), jnp.float32),
            pltpu.VMEM((F,), jnp.float32),
            pltpu.VMEM_SHARED((rows_pad, F), jnp.float32),
            pltpu.SemaphoreType.DMA,
        ],
    )
    def sc_kernel(h1r, ng_h, ns_h, eg_h, es_h, z64_h, ones_h, b1p_h,
                  out, tbl, ng, ns, eg, es, gbuf, sbuf, inv_e, inv_n, zv,
                  onesv, b1v, acc, sem):
        cid = lax.axis_index("c")
        sid = lax.axis_index("s")
        base = sid * (chunks_per_tile * _RB)

        pltpu.sync_copy(ng_h.at[sid], ng)
        pltpu.sync_copy(ns_h.at[sid], ns)
        pltpu.sync_copy(eg_h.at[sid], eg)
        pltpu.sync_copy(es_h.at[sid], es)
        pltpu.sync_copy(z64_h, zv)
        pltpu.sync_copy(ones_h, onesv)
        pltpu.sync_copy(b1p_h, b1v)

        def zero_rows(k, _):
            r0 = base + k * _RB
            pltpu.sync_copy(zv, acc.at[pl.ds(r0, _RB)])
            return 0
        lax.fori_loop(0, chunks_per_tile, zero_rows, 0)
        plsc.subcore_barrier()

        def count_pass(siv):
            def body(j, _):
                pltpu.sync_copy(onesv, acc.at[siv.at[j]], add=True)
                return 0
            lax.fori_loop(0, nch, body, 0)

        lane = lax.iota(jnp.int32, 16)
        lane0 = lane == 0

        def compress_invert(invt):
            def body(k, _):
                r0 = base + k * _RB
                pltpu.sync_copy(acc.at[pl.ds(r0, _RB)], sbuf)

                def row(i, _):
                    v = sbuf[i, pl.ds(0, 16)]
                    inv = jnp.where(v > 0.0, 1.0 / v, 0.0)
                    plsc.store_scatter(
                        invt, [jnp.full((16,), k * _RB + i, jnp.int32)], inv,
                        mask=lane0)
                    return 0
                lax.fori_loop(0, _RB, row, 0)
                pltpu.sync_copy(zv, acc.at[pl.ds(r0, _RB)])
                return 0
            lax.fori_loop(0, chunks_per_tile, body, 0)

        def agg_pass(src_tbl, giv, siv):
            def body(j, _):
                pltpu.async_copy(src_tbl.at[giv.at[j]], gbuf, sem).wait()
                pltpu.sync_copy(gbuf, acc.at[siv.at[j]], add=True)
                return 0
            lax.fori_loop(0, nch, body, 0)

        def scale_rows(inv, add_bias, out_tbl):
            def body(k, _):
                r0 = base + k * _RB
                pltpu.sync_copy(acc.at[pl.ds(r0, _RB)], sbuf)

                def row(i, _):
                    s = plsc.load_gather(
                        inv, [jnp.full((16,), k * _RB + i, jnp.int32)])
                    for q in range(F // 16):
                        v = sbuf[i, pl.ds(q * 16, 16)] * s
                        if add_bias:
                            v = v + b1v[pl.ds(q * 16, 16)]
                        sbuf[i, pl.ds(q * 16, 16)] = v
                    return 0
                lax.fori_loop(0, _RB, row, 0)
                pltpu.sync_copy(sbuf, out_tbl.at[pl.ds(r0, _RB)])
                pltpu.sync_copy(zv, acc.at[pl.ds(r0, _RB)])
                return 0
            lax.fori_loop(0, chunks_per_tile, body, 0)

        h1c = h1r.at[cid]
        tblc = tbl.at[cid]
        outc = out.at[cid]

        count_pass(es)
        plsc.subcore_barrier()
        compress_invert(inv_e)
        plsc.subcore_barrier()
        count_pass(ns)
        plsc.subcore_barrier()
        compress_invert(inv_n)
        plsc.subcore_barrier()

        agg_pass(h1c, ng, es)
        plsc.subcore_barrier()
        scale_rows(inv_e, False, tblc)
        plsc.subcore_barrier()
        agg_pass(tblc, eg, ns)
        plsc.subcore_barrier()
        scale_rows(inv_n, True, tblc)
        plsc.subcore_barrier()
        agg_pass(tblc, ng, es)
        plsc.subcore_barrier()
        scale_rows(inv_e, False, tblc)
        plsc.subcore_barrier()
        agg_pass(tblc, eg, ns)
        plsc.subcore_barrier()
        scale_rows(inv_n, False, outc)

    return sc_kernel


def kernel(x, incident_matrix, ln_pre_g, ln_pre_b, lin1_W, lin1_b, ln1_g,
           ln1_b, conv1_W, conv1_b, conv2_W, conv2_b, ln2_g, ln2_b, lin2_W,
           lin2_b):
    B, N, C = x.shape
    h2 = lin1_W.shape[0]
    h4 = conv1_W.shape[0]
    hidden = conv2_W.shape[0]
    nnz = incident_matrix.shape[1]
    F = (B // _NC) * h4
    BN = 1000

    node = incident_matrix[0].astype(jnp.int32)
    edge = incident_matrix[1].astype(jnp.int32)

    nch = -(-nnz // (_NS * _CHUNK))
    nnz_pad = _NS * nch * _CHUNK
    rows_pad = -(-(N + 1) // (_NS * _RB)) * (_NS * _RB)

    def pad_idx(idx, fill):
        p = jnp.full((nnz_pad - nnz,), fill, dtype=jnp.int32)
        return jnp.concatenate([idx, p]).reshape(_NS, nch, _CHUNK)

    ng = pad_idx(node, 0)
    ns = pad_idx(node, N)
    eg = pad_idx(edge, 0)
    es = pad_idx(edge, N)

    grid1 = (B, N // BN)
    row2d = lambda a: a.reshape(1, -1)
    full = lambda shape: pl.BlockSpec(shape, lambda b, i: (0, 0))
    h1 = pl.pallas_call(
        _tc1_body,
        grid=grid1,
        in_specs=[
            pl.BlockSpec((1, BN, C), lambda b, i: (b, i, 0)),
            full((1, C)), full((1, C)),
            pl.BlockSpec((C, h2), lambda b, i: (0, 0)),
            full((1, h2)), full((1, h2)), full((1, h2)),
            pl.BlockSpec((h2, h4), lambda b, i: (0, 0)),
        ],
        out_specs=pl.BlockSpec((1, BN, h4), lambda b, i: (b, i, 0)),
        out_shape=jax.ShapeDtypeStruct((B, N, h4), jnp.float32),
    )(x, row2d(ln_pre_g), row2d(ln_pre_b), lin1_W.T, row2d(lin1_b),
      row2d(ln1_g), row2d(ln1_b), conv1_W.T)

    h1r = h1.reshape(_NC, B // _NC, N, h4).transpose(0, 2, 1, 3).reshape(
        _NC, N, F)
    b1p = jnp.tile(conv1_b, B // _NC)

    z64 = jnp.zeros((_RB, F), jnp.float32)
    ones64 = jnp.ones((_CHUNK, F), jnp.float32)

    h1p = jnp.zeros((_NC, rows_pad, F), jnp.float32).at[:, :N, :].set(h1r)

    sc = _make_sc_kernel(N, F, nch, rows_pad)
    w_pad, _ = sc(h1p, ng, ns, eg, es, z64, ones64, b1p)

    w8 = w_pad[:, :N, :].reshape(_NC, N, B // _NC, h4).transpose(
        0, 2, 1, 3).reshape(B, N, h4)

    out = pl.pallas_call(
        _tc2_body,
        grid=grid1,
        in_specs=[
            pl.BlockSpec((1, BN, h4), lambda b, i: (b, i, 0)),
            pl.BlockSpec((1, BN, C), lambda b, i: (b, i, 0)),
            pl.BlockSpec((h4, hidden), lambda b, i: (0, 0)),
            full((1, hidden)), full((1, hidden)), full((1, hidden)),
            pl.BlockSpec((hidden, C), lambda b, i: (0, 0)),
            full((1, C)),
        ],
        out_specs=pl.BlockSpec((1, BN, C), lambda b, i: (b, i, 0)),
        out_shape=jax.ShapeDtypeStruct((B, N, C), jnp.float32),
    )(w8, x, conv2_W.T, row2d(conv2_b), row2d(ln2_g), row2d(ln2_b),
      lin2_W.T, row2d(lin2_b))

    return out


# speedup vs baseline: 70.0534x; 1.0214x over previous
import functools

import jax
import jax.numpy as jnp
from jax import lax
from jax.experimental import pallas as pl
from jax.experimental.pallas import tpu as pltpu
from jax.experimental.pallas import tpu_sc as plsc


def _tc1_body(x_ref, lpg_ref, lpb_ref, w1t_ref, b1_ref, l1g_ref, l1b_ref,
              c1wt_ref, o_ref):
    xb = x_ref[0]
    mu = jnp.mean(xb, axis=-1, keepdims=True)
    var = jnp.mean((xb - mu) ** 2, axis=-1, keepdims=True)
    y = (xb - mu) / jnp.sqrt(var + 1e-5) * lpg_ref[0] + lpb_ref[0]
    y = jnp.maximum(y, 0.0)
    z = jnp.dot(y, w1t_ref[...], preferred_element_type=jnp.float32) + b1_ref[0]
    mu2 = jnp.mean(z, axis=-1, keepdims=True)
    var2 = jnp.mean((z - mu2) ** 2, axis=-1, keepdims=True)
    z = (z - mu2) / jnp.sqrt(var2 + 1e-5) * l1g_ref[0] + l1b_ref[0]
    z = jnp.maximum(z, 0.0)
    o_ref[0] = jnp.dot(z, c1wt_ref[...], preferred_element_type=jnp.float32)


def _tc2_body(w_ref, x_ref, c2wt_ref, b2_ref, l2g_ref, l2b_ref, w2t_ref,
              b2l_ref, o_ref):
    wb = w_ref[0]
    o2 = jnp.dot(wb, c2wt_ref[...], preferred_element_type=jnp.float32) + b2_ref[0]
    mu = jnp.mean(o2, axis=-1, keepdims=True)
    var = jnp.mean((o2 - mu) ** 2, axis=-1, keepdims=True)
    t = (o2 - mu) / jnp.sqrt(var + 1e-5) * l2g_ref[0] + l2b_ref[0]
    t = jnp.maximum(t, 0.0)
    y = jnp.dot(t, w2t_ref[...], preferred_element_type=jnp.float32) + b2l_ref[0]
    o_ref[0] = x_ref[0] + y


_NS = 16
_NC = 2
_CHUNK = 128
_RB = 64


def _make_sc_kernel(N, F, nch, rows_pad):
    chunks_per_tile = rows_pad // (_NS * _RB)
    mesh = plsc.VectorSubcoreMesh(core_axis_name="c", subcore_axis_name="s")

    @functools.partial(
        pl.kernel,
        out_type=(jax.ShapeDtypeStruct((_NC, rows_pad, F), jnp.float32),
                  jax.ShapeDtypeStruct((_NC, rows_pad, F), jnp.float32)),
        mesh=mesh,
        compiler_params=pltpu.CompilerParams(
            use_tc_tiling_on_sc=False, needs_layout_passes=False),
        scratch_types=[
            pltpu.VMEM((nch, _CHUNK), jnp.int32),
            pltpu.VMEM((nch, _CHUNK), jnp.int32),
            pltpu.VMEM((_CHUNK, F), jnp.float32),
            pltpu.VMEM((_CHUNK, F), jnp.float32),
            pltpu.VMEM((_CHUNK, F), jnp.float32),
            pltpu.VMEM((_CHUNK, F), jnp.float32),
            pltpu.VMEM((_RB, F), jnp.float32),
            pltpu.VMEM((rows_pad // _NS,), jnp.float32),
            pltpu.VMEM((rows_pad // _NS,), jnp.float32),
            pltpu.VMEM((_RB, F), jnp.float32),
            pltpu.VMEM((_CHUNK, F), jnp.float32),
            pltpu.VMEM((F,), jnp.float32),
            pltpu.VMEM_SHARED((rows_pad, F), jnp.float32),
            pltpu.SemaphoreType.DMA,
        ],
    )
    def sc_kernel(h1r, ni_h, ei_h, z64_h, ones_h, b1p_h,
                  out, tbl, ni, ei, gbuf0, gbuf1, gbuf2, gbuf3,
                  sbuf, inv_e, inv_n, zv, onesv, b1v, acc, sem):
        gbufs = (gbuf0, gbuf1, gbuf2, gbuf3)
        nring = len(gbufs)
        cid = lax.axis_index("c")
        sid = lax.axis_index("s")
        base = sid * (chunks_per_tile * _RB)

        pltpu.sync_copy(ni_h.at[sid], ni)
        pltpu.sync_copy(ei_h.at[sid], ei)
        pltpu.sync_copy(z64_h, zv)
        pltpu.sync_copy(ones_h, onesv)
        pltpu.sync_copy(b1p_h, b1v)

        def zero_rows(k, _):
            r0 = base + k * _RB
            pltpu.sync_copy(zv, acc.at[pl.ds(r0, _RB)])
            return 0
        lax.fori_loop(0, chunks_per_tile, zero_rows, 0)
        plsc.subcore_barrier()

        cwin = 8

        def count_pass(siv):
            def body(j, _):
                pltpu.async_copy(onesv, acc.at[siv.at[j]], sem, add=True)

                @pl.when(j >= cwin)
                def _():
                    pltpu.make_async_copy(onesv, acc.at[siv.at[0]],
                                          sem).wait()
                return 0
            lax.fori_loop(0, nch, body, 0)

            def drain(j, _):
                pltpu.make_async_copy(onesv, acc.at[siv.at[0]], sem).wait()
                return 0
            lax.fori_loop(0, min(cwin, nch), drain, 0)

        lane = lax.iota(jnp.int32, 16)
        lane0 = lane == 0

        def compress_invert(invt):
            def body(k, _):
                r0 = base + k * _RB
                pltpu.sync_copy(acc.at[pl.ds(r0, _RB)], sbuf)

                def row(i, _):
                    v = sbuf[i, pl.ds(0, 16)]
                    inv = jnp.where(v > 0.0, 1.0 / v, 0.0)
                    plsc.store_scatter(
                        invt, [jnp.full((16,), k * _RB + i, jnp.int32)], inv,
                        mask=lane0)
                    return 0
                lax.fori_loop(0, _RB, row, 0)
                pltpu.sync_copy(zv, acc.at[pl.ds(r0, _RB)])
                return 0
            lax.fori_loop(0, chunks_per_tile, body, 0)

        def agg_pass(src_tbl, giv, siv):
            for r in range(nring):
                pltpu.async_copy(src_tbl.at[giv.at[r]], gbufs[r], sem)

            def body(jj, _):
                for r in range(nring):
                    j = jj * nring + r
                    pltpu.make_async_copy(src_tbl.at[giv.at[0]], gbufs[r],
                                          sem).wait()
                    pltpu.sync_copy(gbufs[r], acc.at[siv.at[j]], add=True)

                    @pl.when(j + nring < nch)
                    def _():
                        pltpu.async_copy(src_tbl.at[giv.at[j + nring]],
                                         gbufs[r], sem)
                return 0
            lax.fori_loop(0, nch // nring, body, 0)

        def scale_rows(inv, add_bias, out_tbl):
            def body(k, _):
                r0 = base + k * _RB
                pltpu.sync_copy(acc.at[pl.ds(r0, _RB)], sbuf)

                def row(i, _):
                    s = plsc.load_gather(
                        inv, [jnp.full((16,), k * _RB + i, jnp.int32)])
                    for q in range(F // 16):
                        v = sbuf[i, pl.ds(q * 16, 16)] * s
                        if add_bias:
                            v = v + b1v[pl.ds(q * 16, 16)]
                        sbuf[i, pl.ds(q * 16, 16)] = v
                    return 0
                lax.fori_loop(0, _RB, row, 0)
                pltpu.sync_copy(sbuf, out_tbl.at[pl.ds(r0, _RB)])
                pltpu.sync_copy(zv, acc.at[pl.ds(r0, _RB)])
                return 0
            lax.fori_loop(0, chunks_per_tile, body, 0)

        h1c = h1r.at[cid]
        tblc = tbl.at[cid]
        outc = out.at[cid]

        count_pass(ei)
        plsc.subcore_barrier()
        compress_invert(inv_e)
        plsc.subcore_barrier()
        count_pass(ni)
        plsc.subcore_barrier()
        compress_invert(inv_n)
        plsc.subcore_barrier()

        agg_pass(h1c, ni, ei)
        plsc.subcore_barrier()
        scale_rows(inv_e, False, tblc)
        plsc.subcore_barrier()
        agg_pass(tblc, ei, ni)
        plsc.subcore_barrier()
        scale_rows(inv_n, True, tblc)
        plsc.subcore_barrier()
        agg_pass(tblc, ni, ei)
        plsc.subcore_barrier()
        scale_rows(inv_e, False, tblc)
        plsc.subcore_barrier()
        agg_pass(tblc, ei, ni)
        plsc.subcore_barrier()
        scale_rows(inv_n, False, outc)

    return sc_kernel


def kernel(x, incident_matrix, ln_pre_g, ln_pre_b, lin1_W, lin1_b, ln1_g,
           ln1_b, conv1_W, conv1_b, conv2_W, conv2_b, ln2_g, ln2_b, lin2_W,
           lin2_b):
    B, N, C = x.shape
    h2 = lin1_W.shape[0]
    h4 = conv1_W.shape[0]
    hidden = conv2_W.shape[0]
    nnz = incident_matrix.shape[1]
    F = (B // _NC) * h4
    BN = 1000

    node = incident_matrix[0].astype(jnp.int32)
    edge = incident_matrix[1].astype(jnp.int32)

    nch = -(-nnz // (_NS * _CHUNK))
    nch = -(-nch // 4) * 4
    nnz_pad = _NS * nch * _CHUNK
    rows_pad = -(-(N + 1) // (_NS * _RB)) * (_NS * _RB)

    def pad_idx(idx):
        p = jnp.full((nnz_pad - nnz,), N, dtype=jnp.int32)
        return jnp.concatenate([idx, p]).reshape(_NS, nch, _CHUNK)

    ni = pad_idx(node)
    ei = pad_idx(edge)

    grid1 = (B, N // BN)
    row2d = lambda a: a.reshape(1, -1)
    full = lambda shape: pl.BlockSpec(shape, lambda b, i: (0, 0))
    h1 = pl.pallas_call(
        _tc1_body,
        grid=grid1,
        in_specs=[
            pl.BlockSpec((1, BN, C), lambda b, i: (b, i, 0)),
            full((1, C)), full((1, C)),
            pl.BlockSpec((C, h2), lambda b, i: (0, 0)),
            full((1, h2)), full((1, h2)), full((1, h2)),
            pl.BlockSpec((h2, h4), lambda b, i: (0, 0)),
        ],
        out_specs=pl.BlockSpec((1, BN, h4), lambda b, i: (b, i, 0)),
        out_shape=jax.ShapeDtypeStruct((B, N, h4), jnp.float32),
    )(x, row2d(ln_pre_g), row2d(ln_pre_b), lin1_W.T, row2d(lin1_b),
      row2d(ln1_g), row2d(ln1_b), conv1_W.T)

    h1r = h1.reshape(_NC, B // _NC, N, h4).transpose(0, 2, 1, 3).reshape(
        _NC, N, F)
    b1p = jnp.tile(conv1_b, B // _NC)

    z64 = jnp.zeros((_RB, F), jnp.float32)
    ones64 = jnp.ones((_CHUNK, F), jnp.float32)

    h1p = jnp.zeros((_NC, rows_pad, F), jnp.float32).at[:, :N, :].set(h1r)

    sc = _make_sc_kernel(N, F, nch, rows_pad)
    w_pad, _ = sc(h1p, ni, ei, z64, ones64, b1p)

    w8 = w_pad[:, :N, :].reshape(_NC, N, B // _NC, h4).transpose(
        0, 2, 1, 3).reshape(B, N, h4)

    out = pl.pallas_call(
        _tc2_body,
        grid=grid1,
        in_specs=[
            pl.BlockSpec((1, BN, h4), lambda b, i: (b, i, 0)),
            pl.BlockSpec((1, BN, C), lambda b, i: (b, i, 0)),
            pl.BlockSpec((h4, hidden), lambda b, i: (0, 0)),
            full((1, hidden)), full((1, hidden)), full((1, hidden)),
            pl.BlockSpec((hidden, C), lambda b, i: (0, 0)),
            full((1, C)),
        ],
        out_specs=pl.BlockSpec((1, BN, C), lambda b, i: (b, i, 0)),
        out_shape=jax.ShapeDtypeStruct((B, N, C), jnp.float32),
    )(w8, x, conv2_W.T, row2d(conv2_b), row2d(ln2_g), row2d(ln2_b),
      lin2_W.T, row2d(lin2_b))

    return out
